# speedup vs baseline: 5.7329x; 5.7329x over previous
import jax
import jax.numpy as jnp
from jax.experimental import pallas as pl

B = 1024
L_TRAJ = 100
L_ENV = 32
L_TOT = L_TRAJ + L_ENV
D = 512
PATCH_SQ = 1024
T_VOCAB = 21
S_VOCAB = 38
V_TOT = T_VOCAB + S_VOCAB
EPS = 1e-12

BB = 32


def _ln(x, gamma, beta):
    m = jnp.mean(x, axis=-1, keepdims=True)
    d = x - m
    v = jnp.mean(d * d, axis=-1, keepdims=True)
    return d * jax.lax.rsqrt(v + EPS) * gamma + beta


def _body(x0_ref, x1_ref, t_tr_ref, s_tr_ref, env_ref, t_ev_ref, s_ev_ref,
          w_sp_ref, b_sp_ref, w_env_ref, b_env_ref, tbl_ref, gamma_ref,
          beta_ref, out_ref):
    gamma = gamma_ref[...]
    beta = beta_ref[...]
    tbl = tbl_ref[...]

    def emb(t_ref, s_ref, n):
        col = jax.lax.broadcasted_iota(jnp.int32, (BB, n, V_TOT), 2)
        t = t_ref[...][:, :, None]
        s = s_ref[...][:, :, None] + T_VOCAB
        oh = jnp.logical_or(col == t, col == s).astype(jnp.bfloat16)
        return jax.lax.dot_general(
            oh, tbl, (((2,), (0,)), ((), ())),
            preferred_element_type=jnp.float32)

    x0 = x0_ref[...]
    x1 = x1_ref[...]
    sp = (x0[:, :, None] * w_sp_ref[0][None, None, :]
          + x1[:, :, None] * w_sp_ref[1][None, None, :]
          + b_sp_ref[...][None, None, :])
    sp = jnp.maximum(sp, 0.0)
    traj = sp + emb(t_tr_ref, s_tr_ref, L_TRAJ)
    out_ref[:, :L_TRAJ, :] = _ln(traj, gamma, beta)

    env = env_ref[...].astype(jnp.bfloat16).reshape(BB * L_ENV, PATCH_SQ)
    ev = jax.lax.dot_general(
        env, w_env_ref[...], (((1,), (0,)), ((), ())),
        preferred_element_type=jnp.float32)
    ev = jnp.maximum(ev.reshape(BB, L_ENV, D) + b_env_ref[...][None, None, :],
                     0.0)
    scene = ev + emb(t_ev_ref, s_ev_ref, L_ENV)
    out_ref[:, L_TRAJ:, :] = _ln(scene, gamma, beta)


@jax.jit
def kernel(spatial_ids, temporal_ids, segment_ids, env_spatial_ids,
           env_temporal_ids, env_segment_ids, W_sp, b_sp, temporal_table,
           segment_table, W_env, b_env, ln_gamma, ln_beta):
    x0 = spatial_ids[:, :, 0]
    x1 = spatial_ids[:, :, 1]
    tbl = jnp.concatenate([temporal_table, segment_table], axis=0)
    tbl = tbl.astype(jnp.bfloat16)
    w_env = W_env.astype(jnp.bfloat16)

    grid = (B // BB,)
    bspec = lambda shape: pl.BlockSpec(
        shape, lambda i: (i,) + (0,) * (len(shape) - 1))
    full = lambda shape: pl.BlockSpec(shape, lambda i: (0,) * len(shape))

    return pl.pallas_call(
        _body,
        grid=grid,
        in_specs=[
            bspec((BB, L_TRAJ)),
            bspec((BB, L_TRAJ)),
            bspec((BB, L_TRAJ)),
            bspec((BB, L_TRAJ)),
            bspec((BB, L_ENV, PATCH_SQ)),
            bspec((BB, L_ENV)),
            bspec((BB, L_ENV)),
            full((2, D)),
            full((D,)),
            full((PATCH_SQ, D)),
            full((D,)),
            full((V_TOT, D)),
            full((D,)),
            full((D,)),
        ],
        out_specs=bspec((BB, L_TOT, D)),
        out_shape=jax.ShapeDtypeStruct((B, L_TOT, D), jnp.float32),
    )(x0, x1, temporal_ids, segment_ids, env_spatial_ids, env_temporal_ids,
      env_segment_ids, W_sp, b_sp, w_env, b_env, tbl, ln_gamma, ln_beta)
